# Initial kernel scaffold; baseline (speedup 1.0000x reference)
import jax
import jax.numpy as jnp
from jax.experimental import pallas as pl


def kernel(x, edge_index, W, att_src, att_dst, conv_bias, fc_W, fc_b, ln_gamma, ln_beta, gfc_W, gfc_b):
    raise NotImplementedError("write your pallas kernel here")


# speedup vs baseline: 173.5089x; 173.5089x over previous
import functools

import jax
import jax.numpy as jnp
from jax import lax
from jax.experimental import pallas as pl
from jax.experimental.pallas import tpu as pltpu
from jax.experimental.pallas import tpu_sc as plsc

N = 10000
E = 320000
D = 128
H = 8
HD = 16

NP = 8
NX = N + NP
A = E + N
NW = 32
B = 64
NCH = 162
C = NCH * B
A_PAD = NW * C
PAD = A_PAD - A
NA = 10240
TPT = NA // 16
ZR = 128


def _prep_body(x_ref, wc_ref, as_ref, ad_ref, hx_ref, asx_ref, adx_ref, mg_ref):
    hx = jnp.dot(x_ref[:], wc_ref[:], preferred_element_type=jnp.float32)
    a_s = jnp.dot(hx, as_ref[:], preferred_element_type=jnp.float32)
    a_d = jnp.dot(hx, ad_ref[:], preferred_element_type=jnp.float32)
    hx_ref[:] = jnp.concatenate([hx, jnp.zeros((NP, D), jnp.float32)], axis=0)
    neg = jnp.full((NP, 2 * H), -1e30, jnp.float32)
    z = jnp.zeros((N, H), jnp.float32)
    asx_ref[:] = jnp.concatenate(
        [jnp.concatenate([a_s, z], axis=1), neg], axis=0)
    adx_ref[:] = jnp.concatenate(
        [jnp.concatenate([a_d, z], axis=1), jnp.zeros((NP, 2 * H))], axis=0)
    m = jnp.max(a_s, axis=0) + jnp.max(a_d, axis=0)
    mg = jnp.where(m >= 0, m, 0.2 * m)
    mg_ref[:] = jnp.concatenate([mg, mg]).reshape(1, 2 * H)


def _prep(x, w_cat, a_src_m, a_dst_m):
    return pl.pallas_call(
        _prep_body,
        out_shape=[
            jax.ShapeDtypeStruct((NX, D), jnp.float32),
            jax.ShapeDtypeStruct((NX, 2 * H), jnp.float32),
            jax.ShapeDtypeStruct((NX, 2 * H), jnp.float32),
            jax.ShapeDtypeStruct((1, 2 * H), jnp.float32),
        ],
    )(x, w_cat, a_src_m, a_dst_m)


_MESH = plsc.VectorSubcoreMesh(core_axis_name="c", subcore_axis_name="s")


@functools.partial(
    pl.kernel,
    out_type=[
        jax.ShapeDtypeStruct((2, NA, D), jnp.float32),
        jax.ShapeDtypeStruct((2, NA, 2 * H), jnp.float32),
    ],
    mesh=_MESH,
    compiler_params=pltpu.CompilerParams(use_tc_tiling_on_sc=False),
    scratch_types=[
        pltpu.VMEM((NCH, B), jnp.int32),
        pltpu.VMEM((NCH, B), jnp.int32),
        pltpu.VMEM((B, 2 * H), jnp.float32),
        pltpu.VMEM((B, 2 * H), jnp.float32),
        pltpu.VMEM((B, 2 * H), jnp.float32),
        pltpu.VMEM((B, D), jnp.float32),
        pltpu.VMEM((16,), jnp.float32),
        pltpu.SemaphoreType.DMA,
        pltpu.SemaphoreType.DMA,
        pltpu.SemaphoreType.DMA,
        pltpu.VMEM_SHARED((NA, D), jnp.float32),
        pltpu.VMEM_SHARED((NA, 2 * H), jnp.float32),
    ],
)
def _edges(hx_hbm, asx_hbm, adx_hbm, mg_hbm, src_hbm, dst_hbm,
           outp_hbm, denp_hbm,
           srcv, dstv, asb, adb, wb, hxb, mgv, sem1, sem2, sem3,
           out_sh, den_sh):
    cid = lax.axis_index("c")
    sid = lax.axis_index("s")
    wid = sid * 2 + cid

    pltpu.sync_copy(src_hbm.at[wid], srcv)
    pltpu.sync_copy(dst_hbm.at[wid], dstv)
    pltpu.sync_copy(mg_hbm.at[0], mgv)

    z16 = jnp.zeros((16,), jnp.float32)

    def zero_row(i, carry):
        for j in range(D // 16):
            hxb[i, pl.ds(16 * j, 16)] = z16
        wb[i, :] = z16
        return carry

    lax.fori_loop(0, B, zero_row, 0)
    for k in range(TPT // B):
        base = sid * TPT + k * B
        pltpu.sync_copy(hxb, out_sh.at[pl.ds(base, B)])
        pltpu.sync_copy(wb, den_sh.at[pl.ds(base, B)])
    plsc.subcore_barrier()

    mg = mgv[:]

    def chunk(j, carry):
        d1 = pltpu.async_copy(asx_hbm.at[srcv.at[j]], asb, sem1)
        d2 = pltpu.async_copy(adx_hbm.at[dstv.at[j]], adb, sem2)
        d3 = pltpu.async_copy(hx_hbm.at[srcv.at[j]], hxb, sem3)
        d1.wait()
        d2.wait()

        def wrow(i, c2):
            e0 = asb[i, :] + adb[i, :]
            wb[i, :] = jnp.exp(jnp.where(e0 >= 0, e0, 0.2 * e0) - mg)
            return c2

        lax.fori_loop(0, B, wrow, 0)
        pltpu.sync_copy(wb, den_sh.at[dstv.at[j]], add=True)
        d3.wait()

        def scale(i, c2):
            wv = wb[i, :]
            for h in range(H):
                s = wv[h]
                hxb[i, pl.ds(16 * h, 16)] = hxb[i, pl.ds(16 * h, 16)] * s
            return c2

        lax.fori_loop(0, B, scale, 0)
        pltpu.sync_copy(hxb, out_sh.at[dstv.at[j]], add=True)
        return carry

    lax.fori_loop(0, NCH, chunk, 0)
    plsc.subcore_barrier()

    base = sid * TPT
    pltpu.sync_copy(out_sh.at[pl.ds(base, TPT)],
                    outp_hbm.at[cid, pl.ds(base, TPT)])
    pltpu.sync_copy(den_sh.at[pl.ds(base, TPT)],
                    denp_hbm.at[cid, pl.ds(base, TPT)])


def _tail_body(outp_ref, denp_ref, ssel_ref, bias_ref, fcw_ref, fcb_ref,
               g_ref, b_ref, gfcw_ref, gfcb_ref, o_ref):
    num = outp_ref[0, 0:N, :] + outp_ref[1, 0:N, :]
    den8 = denp_ref[0, 0:N, 0:H] + denp_ref[1, 0:N, 0:H]
    d128 = jnp.dot(1.0 / (den8 + 1e-16), ssel_ref[:],
                   preferred_element_type=jnp.float32)
    xl = num * d128 + bias_ref[:]
    t = lax.dot_general(xl, fcw_ref[:], (((1,), (1,)), ((), ())),
                        preferred_element_type=jnp.float32) + fcb_ref[:]
    sa = jnp.where(t >= 0, t, 0.01 * t)
    sa = jnp.exp(sa - jnp.max(sa, axis=-1, keepdims=True))
    sa = sa / jnp.sum(sa, axis=-1, keepdims=True)
    xl = xl * sa
    xl = jnp.where(xl >= 0, xl, 0.2 * xl)
    xl = lax.dot_general(xl, fcw_ref[:], (((1,), (1,)), ((), ())),
                         preferred_element_type=jnp.float32) + fcb_ref[:]
    mu = jnp.mean(xl, axis=-1, keepdims=True)
    var = jnp.mean((xl - mu) ** 2, axis=-1, keepdims=True)
    xl = (xl - mu) / jnp.sqrt(var + 1e-5) * g_ref[:] + b_ref[:]
    nrm = jnp.sqrt(jnp.sum(xl * xl, axis=1, keepdims=True))
    xl = xl / jnp.maximum(nrm, 1e-12)
    xg = jnp.mean(xl, axis=0, keepdims=True)
    ga = lax.dot_general(xg, gfcw_ref[:], (((1,), (1,)), ((), ())),
                         preferred_element_type=jnp.float32) + gfcb_ref[:]
    ga = jnp.maximum(ga, 0.0)
    ga = jnp.exp(ga - jnp.max(ga, axis=-1, keepdims=True))
    ga = ga / jnp.sum(ga, axis=-1, keepdims=True)
    o_ref[:] = xl * ga


def _tail(outp, denp, ssel, bias128, fc_W, fc_b2, g2, b2, gfc_W, gfc_b2):
    return pl.pallas_call(
        _tail_body,
        out_shape=jax.ShapeDtypeStruct((N, D), jnp.float32),
    )(outp, denp, ssel, bias128, fc_W, fc_b2, g2, b2, gfc_W, gfc_b2)


def kernel(x, edge_index, W, att_src, att_dst, conv_bias, fc_W, fc_b,
           ln_gamma, ln_beta, gfc_W, gfc_b):
    f32 = jnp.float32
    w_cat = jnp.transpose(W, (1, 0, 2)).reshape(D, D)
    a_src_m = (jnp.eye(H, dtype=f32)[:, None, :]
               * att_src[:, :, None]).reshape(D, H)
    a_dst_m = (jnp.eye(H, dtype=f32)[:, None, :]
               * att_dst[:, :, None]).reshape(D, H)
    ssel = jnp.repeat(jnp.eye(H, dtype=f32), HD, axis=1)
    bias128 = conv_bias.reshape(1, D)

    loops = jnp.arange(N, dtype=jnp.int32)
    pad_i = jnp.arange(PAD, dtype=jnp.int32)
    src_all = jnp.concatenate(
        [edge_index[0], loops, N + (pad_i % NP)]).reshape(NW, NCH, B)
    dst_all = jnp.concatenate(
        [edge_index[1], loops, pad_i % N]).reshape(NW, NCH, B)

    hx_x, asx, adx, mg2 = _prep(x, w_cat, a_src_m, a_dst_m)
    outp, denp = _edges(hx_x, asx, adx, mg2, src_all, dst_all)
    return _tail(outp, denp, ssel, bias128, fc_W,
                 fc_b.reshape(1, D), ln_gamma.reshape(1, D),
                 ln_beta.reshape(1, D), gfc_W, gfc_b.reshape(1, D))


# speedup vs baseline: 188.3678x; 1.0856x over previous
import functools

import jax
import jax.numpy as jnp
from jax import lax
from jax.experimental import pallas as pl
from jax.experimental.pallas import tpu as pltpu
from jax.experimental.pallas import tpu_sc as plsc

N = 10000
E = 320000
D = 128
H = 8
HD = 16

NP = 8
NX = N + NP
A = E + N
NW = 32
B = 96
NCH = 108
C = NCH * B
A_PAD = NW * C
PAD = A_PAD - A
NA = 10240
TPT = NA // 16
ZR = 128


def _prep_body(x_ref, wc_ref, as_ref, ad_ref, hx_ref, asx_ref, adx_ref, mg_ref):
    hx = jnp.dot(x_ref[:], wc_ref[:], preferred_element_type=jnp.float32)
    a_s = jnp.dot(hx, as_ref[:], preferred_element_type=jnp.float32)
    a_d = jnp.dot(hx, ad_ref[:], preferred_element_type=jnp.float32)
    hx_ref[:] = jnp.concatenate([hx, jnp.zeros((NP, D), jnp.float32)], axis=0)
    neg = jnp.full((NP, 2 * H), -1e30, jnp.float32)
    z = jnp.zeros((N, H), jnp.float32)
    asx_ref[:] = jnp.concatenate(
        [jnp.concatenate([a_s, z], axis=1), neg], axis=0)
    adx_ref[:] = jnp.concatenate(
        [jnp.concatenate([a_d, z], axis=1), jnp.zeros((NP, 2 * H))], axis=0)
    m = jnp.max(a_s, axis=0) + jnp.max(a_d, axis=0)
    mg = jnp.where(m >= 0, m, 0.2 * m)
    mg_ref[:] = jnp.concatenate([mg, mg]).reshape(1, 2 * H)


def _prep(x, w_cat, a_src_m, a_dst_m):
    return pl.pallas_call(
        _prep_body,
        out_shape=[
            jax.ShapeDtypeStruct((NX, D), jnp.float32),
            jax.ShapeDtypeStruct((NX, 2 * H), jnp.float32),
            jax.ShapeDtypeStruct((NX, 2 * H), jnp.float32),
            jax.ShapeDtypeStruct((1, 2 * H), jnp.float32),
        ],
    )(x, w_cat, a_src_m, a_dst_m)


_MESH = plsc.VectorSubcoreMesh(core_axis_name="c", subcore_axis_name="s")


@functools.partial(
    pl.kernel,
    out_type=[
        jax.ShapeDtypeStruct((2, NA, D), jnp.float32),
        jax.ShapeDtypeStruct((2, NA, 2 * H), jnp.float32),
    ],
    mesh=_MESH,
    compiler_params=pltpu.CompilerParams(use_tc_tiling_on_sc=False),
    scratch_types=[
        pltpu.VMEM((NCH, B), jnp.int32),
        pltpu.VMEM((NCH, B), jnp.int32),
        pltpu.VMEM((B, 2 * H), jnp.float32),
        pltpu.VMEM((B, 2 * H), jnp.float32),
        pltpu.VMEM((B, 2 * H), jnp.float32),
        pltpu.VMEM((B, D), jnp.float32),
        pltpu.VMEM((16,), jnp.float32),
        pltpu.SemaphoreType.DMA,
        pltpu.SemaphoreType.DMA,
        pltpu.SemaphoreType.DMA,
        pltpu.VMEM_SHARED((NA, D), jnp.float32),
        pltpu.VMEM_SHARED((NA, 2 * H), jnp.float32),
    ],
)
def _edges(hx_hbm, asx_hbm, adx_hbm, mg_hbm, src_hbm, dst_hbm,
           outp_hbm, denp_hbm,
           srcv, dstv, asb, adb, wb, hxb, mgv, sem1, sem2, sem3,
           out_sh, den_sh):
    cid = lax.axis_index("c")
    sid = lax.axis_index("s")
    wid = sid * 2 + cid

    pltpu.sync_copy(src_hbm.at[wid], srcv)
    pltpu.sync_copy(dst_hbm.at[wid], dstv)
    pltpu.sync_copy(mg_hbm.at[0], mgv)

    z16 = jnp.zeros((16,), jnp.float32)

    def zero_row(i, carry):
        for j in range(D // 16):
            hxb[i, pl.ds(16 * j, 16)] = z16
        wb[i, :] = z16
        return carry

    lax.fori_loop(0, B, zero_row, 0)
    for k in range(TPT // B):
        base = sid * TPT + k * B
        pltpu.sync_copy(hxb, out_sh.at[pl.ds(base, B)])
        pltpu.sync_copy(wb, den_sh.at[pl.ds(base, B)])
    plsc.subcore_barrier()

    mg = mgv[:]

    def chunk(j, carry):
        d1 = pltpu.async_copy(asx_hbm.at[srcv.at[j]], asb, sem1)
        d2 = pltpu.async_copy(adx_hbm.at[dstv.at[j]], adb, sem2)
        d3 = pltpu.async_copy(hx_hbm.at[srcv.at[j]], hxb, sem3)
        d1.wait()
        d2.wait()

        def wrow(i, c2):
            e0 = asb[i, :] + adb[i, :]
            wb[i, :] = jnp.exp(jnp.where(e0 >= 0, e0, 0.2 * e0) - mg)
            return c2

        lax.fori_loop(0, B, wrow, 0)
        pltpu.sync_copy(wb, den_sh.at[dstv.at[j]], add=True)
        d3.wait()

        def scale(i, c2):
            wv = wb[i, :]
            for h in range(H):
                s = wv[h]
                hxb[i, pl.ds(16 * h, 16)] = hxb[i, pl.ds(16 * h, 16)] * s
            return c2

        lax.fori_loop(0, B, scale, 0)
        pltpu.sync_copy(hxb, out_sh.at[dstv.at[j]], add=True)
        return carry

    lax.fori_loop(0, NCH, chunk, 0)
    plsc.subcore_barrier()

    base = sid * TPT
    pltpu.sync_copy(out_sh.at[pl.ds(base, TPT)],
                    outp_hbm.at[cid, pl.ds(base, TPT)])
    pltpu.sync_copy(den_sh.at[pl.ds(base, TPT)],
                    denp_hbm.at[cid, pl.ds(base, TPT)])


def _tail_body(outp_ref, denp_ref, ssel_ref, bias_ref, fcw_ref, fcb_ref,
               g_ref, b_ref, gfcw_ref, gfcb_ref, o_ref):
    num = outp_ref[0, 0:N, :] + outp_ref[1, 0:N, :]
    den8 = denp_ref[0, 0:N, 0:H] + denp_ref[1, 0:N, 0:H]
    d128 = jnp.dot(1.0 / (den8 + 1e-16), ssel_ref[:],
                   preferred_element_type=jnp.float32)
    xl = num * d128 + bias_ref[:]
    t = lax.dot_general(xl, fcw_ref[:], (((1,), (1,)), ((), ())),
                        preferred_element_type=jnp.float32) + fcb_ref[:]
    sa = jnp.where(t >= 0, t, 0.01 * t)
    sa = jnp.exp(sa - jnp.max(sa, axis=-1, keepdims=True))
    sa = sa / jnp.sum(sa, axis=-1, keepdims=True)
    xl = xl * sa
    xl = jnp.where(xl >= 0, xl, 0.2 * xl)
    xl = lax.dot_general(xl, fcw_ref[:], (((1,), (1,)), ((), ())),
                         preferred_element_type=jnp.float32) + fcb_ref[:]
    mu = jnp.mean(xl, axis=-1, keepdims=True)
    var = jnp.mean((xl - mu) ** 2, axis=-1, keepdims=True)
    xl = (xl - mu) / jnp.sqrt(var + 1e-5) * g_ref[:] + b_ref[:]
    nrm = jnp.sqrt(jnp.sum(xl * xl, axis=1, keepdims=True))
    xl = xl / jnp.maximum(nrm, 1e-12)
    xg = jnp.mean(xl, axis=0, keepdims=True)
    ga = lax.dot_general(xg, gfcw_ref[:], (((1,), (1,)), ((), ())),
                         preferred_element_type=jnp.float32) + gfcb_ref[:]
    ga = jnp.maximum(ga, 0.0)
    ga = jnp.exp(ga - jnp.max(ga, axis=-1, keepdims=True))
    ga = ga / jnp.sum(ga, axis=-1, keepdims=True)
    o_ref[:] = xl * ga


def _tail(outp, denp, ssel, bias128, fc_W, fc_b2, g2, b2, gfc_W, gfc_b2):
    return pl.pallas_call(
        _tail_body,
        out_shape=jax.ShapeDtypeStruct((N, D), jnp.float32),
    )(outp, denp, ssel, bias128, fc_W, fc_b2, g2, b2, gfc_W, gfc_b2)


def kernel(x, edge_index, W, att_src, att_dst, conv_bias, fc_W, fc_b,
           ln_gamma, ln_beta, gfc_W, gfc_b):
    f32 = jnp.float32
    w_cat = jnp.transpose(W, (1, 0, 2)).reshape(D, D)
    a_src_m = (jnp.eye(H, dtype=f32)[:, None, :]
               * att_src[:, :, None]).reshape(D, H)
    a_dst_m = (jnp.eye(H, dtype=f32)[:, None, :]
               * att_dst[:, :, None]).reshape(D, H)
    ssel = jnp.repeat(jnp.eye(H, dtype=f32), HD, axis=1)
    bias128 = conv_bias.reshape(1, D)

    loops = jnp.arange(N, dtype=jnp.int32)
    pad_i = jnp.arange(PAD, dtype=jnp.int32)
    src_all = jnp.concatenate(
        [edge_index[0], loops, N + (pad_i % NP)]).reshape(NW, NCH, B)
    dst_all = jnp.concatenate(
        [edge_index[1], loops, pad_i % N]).reshape(NW, NCH, B)

    hx_x, asx, adx, mg2 = _prep(x, w_cat, a_src_m, a_dst_m)
    outp, denp = _edges(hx_x, asx, adx, mg2, src_all, dst_all)
    return _tail(outp, denp, ssel, bias128, fc_W,
                 fc_b.reshape(1, D), ln_gamma.reshape(1, D),
                 ln_beta.reshape(1, D), gfc_W, gfc_b.reshape(1, D))


# speedup vs baseline: 244.7982x; 1.2996x over previous
import functools

import jax
import jax.numpy as jnp
from jax import lax
from jax.experimental import pallas as pl
from jax.experimental.pallas import tpu as pltpu
from jax.experimental.pallas import tpu_sc as plsc

N = 10000
E = 320000
D = 128
H = 8
HD = 16

NP = 8
NX = N + NP
A = E + N
NW = 32
B = 96
NCH = 108
C = NCH * B
A_PAD = NW * C
PAD = A_PAD - A
NA = 10240
TPT = NA // 16
ZR = 128


def _prep_body(x_ref, wc_ref, as_ref, ad_ref, hx_ref, asx_ref, adx_ref, mg_ref):
    hx = jnp.dot(x_ref[:], wc_ref[:], preferred_element_type=jnp.float32)
    a_s = jnp.dot(hx, as_ref[:], preferred_element_type=jnp.float32)
    a_d = jnp.dot(hx, ad_ref[:], preferred_element_type=jnp.float32)
    hx_ref[:] = jnp.concatenate([hx, jnp.zeros((NP, D), jnp.float32)], axis=0)
    neg = jnp.full((NP, 2 * H), -1e30, jnp.float32)
    z = jnp.zeros((N, H), jnp.float32)
    asx_ref[:] = jnp.concatenate(
        [jnp.concatenate([a_s, z], axis=1), neg], axis=0)
    adx_ref[:] = jnp.concatenate(
        [jnp.concatenate([a_d, z], axis=1), jnp.zeros((NP, 2 * H))], axis=0)
    m = jnp.max(a_s, axis=0) + jnp.max(a_d, axis=0)
    mg = jnp.where(m >= 0, m, 0.2 * m)
    mg_ref[:] = jnp.concatenate([mg, mg]).reshape(1, 2 * H)


def _prep(x, w_cat, a_src_m, a_dst_m):
    return pl.pallas_call(
        _prep_body,
        out_shape=[
            jax.ShapeDtypeStruct((NX, D), jnp.float32),
            jax.ShapeDtypeStruct((NX, 2 * H), jnp.float32),
            jax.ShapeDtypeStruct((NX, 2 * H), jnp.float32),
            jax.ShapeDtypeStruct((1, 2 * H), jnp.float32),
        ],
    )(x, w_cat, a_src_m, a_dst_m)


_MESH = plsc.VectorSubcoreMesh(core_axis_name="c", subcore_axis_name="s")


@functools.partial(
    pl.kernel,
    out_type=[
        jax.ShapeDtypeStruct((2, NA, D), jnp.float32),
        jax.ShapeDtypeStruct((2, NA, 2 * H), jnp.float32),
    ],
    mesh=_MESH,
    compiler_params=pltpu.CompilerParams(use_tc_tiling_on_sc=False),
    scratch_types=[
        [pltpu.VMEM((B,), jnp.int32) for _ in range(3)],
        [pltpu.VMEM((B,), jnp.int32) for _ in range(3)],
        [pltpu.VMEM((B, 2 * H), jnp.float32) for _ in range(2)],
        [pltpu.VMEM((B, 2 * H), jnp.float32) for _ in range(2)],
        [pltpu.VMEM((B, 2 * H), jnp.float32) for _ in range(2)],
        [pltpu.VMEM((B, D), jnp.float32) for _ in range(2)],
        pltpu.VMEM((16,), jnp.float32),
        [pltpu.SemaphoreType.DMA for _ in range(3)],
        [pltpu.SemaphoreType.DMA for _ in range(2)],
        [pltpu.SemaphoreType.DMA for _ in range(2)],
        [pltpu.SemaphoreType.DMA for _ in range(2)],
        pltpu.VMEM_SHARED((NA, D), jnp.float32),
        pltpu.VMEM_SHARED((NA, 2 * H), jnp.float32),
    ],
)
def _edges(hx_hbm, asx_hbm, adx_hbm, mg_hbm, src_hbm, dst_hbm,
           outp_hbm, denp_hbm,
           isr, idr, asb, adb, wb, hxb, mgv, si, sa, sd, sh,
           out_sh, den_sh):
    cid = lax.axis_index("c")
    sid = lax.axis_index("s")
    wid = sid * 2 + cid

    pltpu.sync_copy(mg_hbm.at[0], mgv)

    z16 = jnp.zeros((16,), jnp.float32)

    def zero_row(i, carry):
        for j in range(D // 16):
            hxb[0][i, pl.ds(16 * j, 16)] = z16
        wb[0][i, :] = z16
        return carry

    lax.fori_loop(0, B, zero_row, 0)
    for k in range(TPT // B):
        base = sid * TPT + k * B
        pltpu.sync_copy(hxb[0], out_sh.at[pl.ds(base, B)])
        pltpu.sync_copy(wb[0], den_sh.at[pl.ds(base, B)])
    plsc.subcore_barrier()

    mg = mgv[:]

    def issue_idx(j, p):
        pltpu.async_copy(src_hbm.at[wid, j], isr[p], si[p])
        pltpu.async_copy(dst_hbm.at[wid, j], idr[p], si[p])

    def wait_idx(j, p):
        pltpu.make_async_copy(src_hbm.at[wid, j], isr[p], si[p]).wait()
        pltpu.make_async_copy(dst_hbm.at[wid, j], idr[p], si[p]).wait()

    def issue_data(p3, p2):
        pltpu.async_copy(asx_hbm.at[isr[p3]], asb[p2], sa[p2])
        pltpu.async_copy(adx_hbm.at[idr[p3]], adb[p2], sd[p2])
        pltpu.async_copy(hx_hbm.at[isr[p3]], hxb[p2], sh[p2])

    def wait_data(p3, p2):
        pltpu.make_async_copy(asx_hbm.at[isr[p3]], asb[p2], sa[p2]).wait()
        pltpu.make_async_copy(adx_hbm.at[idr[p3]], adb[p2], sd[p2]).wait()
        pltpu.make_async_copy(hx_hbm.at[isr[p3]], hxb[p2], sh[p2]).wait()

    issue_idx(0, 0)
    wait_idx(0, 0)
    issue_data(0, 0)
    issue_idx(1, 1)

    def group(g, carry):
        for u in range(6):
            jj = g * 6 + u
            p2, p3 = u % 2, u % 3
            q2, q3 = (u + 1) % 2, (u + 1) % 3
            r3 = (u + 2) % 3

            @pl.when(jj + 1 < NCH)
            def _():
                wait_idx(jj + 1, q3)
                issue_data(q3, q2)

            @pl.when(jj + 2 < NCH)
            def _():
                issue_idx(jj + 2, r3)

            wait_data(p3, p2)

            def wrow(i, c2):
                e0 = asb[p2][i, :] + adb[p2][i, :]
                wb[p2][i, :] = jnp.exp(jnp.where(e0 >= 0, e0, 0.2 * e0) - mg)
                return c2

            lax.fori_loop(0, B, wrow, 0)
            pltpu.sync_copy(wb[p2], den_sh.at[idr[p3]], add=True)

            def scale(i, c2):
                wv = wb[p2][i, :]
                for h in range(H):
                    s = wv[h]
                    hxb[p2][i, pl.ds(16 * h, 16)] = (
                        hxb[p2][i, pl.ds(16 * h, 16)] * s)
                return c2

            lax.fori_loop(0, B, scale, 0)
            pltpu.sync_copy(hxb[p2], out_sh.at[idr[p3]], add=True)
        return carry

    lax.fori_loop(0, NCH // 6, group, 0)
    plsc.subcore_barrier()

    base = sid * TPT
    pltpu.sync_copy(out_sh.at[pl.ds(base, TPT)],
                    outp_hbm.at[cid, pl.ds(base, TPT)])
    pltpu.sync_copy(den_sh.at[pl.ds(base, TPT)],
                    denp_hbm.at[cid, pl.ds(base, TPT)])


def _tail_body(outp_ref, denp_ref, ssel_ref, bias_ref, fcw_ref, fcb_ref,
               g_ref, b_ref, gfcw_ref, gfcb_ref, o_ref):
    num = outp_ref[0, 0:N, :] + outp_ref[1, 0:N, :]
    den8 = denp_ref[0, 0:N, 0:H] + denp_ref[1, 0:N, 0:H]
    d128 = jnp.dot(1.0 / (den8 + 1e-16), ssel_ref[:],
                   preferred_element_type=jnp.float32)
    xl = num * d128 + bias_ref[:]
    t = lax.dot_general(xl, fcw_ref[:], (((1,), (1,)), ((), ())),
                        preferred_element_type=jnp.float32) + fcb_ref[:]
    sa = jnp.where(t >= 0, t, 0.01 * t)
    sa = jnp.exp(sa - jnp.max(sa, axis=-1, keepdims=True))
    sa = sa / jnp.sum(sa, axis=-1, keepdims=True)
    xl = xl * sa
    xl = jnp.where(xl >= 0, xl, 0.2 * xl)
    xl = lax.dot_general(xl, fcw_ref[:], (((1,), (1,)), ((), ())),
                         preferred_element_type=jnp.float32) + fcb_ref[:]
    mu = jnp.mean(xl, axis=-1, keepdims=True)
    var = jnp.mean((xl - mu) ** 2, axis=-1, keepdims=True)
    xl = (xl - mu) / jnp.sqrt(var + 1e-5) * g_ref[:] + b_ref[:]
    nrm = jnp.sqrt(jnp.sum(xl * xl, axis=1, keepdims=True))
    xl = xl / jnp.maximum(nrm, 1e-12)
    xg = jnp.mean(xl, axis=0, keepdims=True)
    ga = lax.dot_general(xg, gfcw_ref[:], (((1,), (1,)), ((), ())),
                         preferred_element_type=jnp.float32) + gfcb_ref[:]
    ga = jnp.maximum(ga, 0.0)
    ga = jnp.exp(ga - jnp.max(ga, axis=-1, keepdims=True))
    ga = ga / jnp.sum(ga, axis=-1, keepdims=True)
    o_ref[:] = xl * ga


def _tail(outp, denp, ssel, bias128, fc_W, fc_b2, g2, b2, gfc_W, gfc_b2):
    return pl.pallas_call(
        _tail_body,
        out_shape=jax.ShapeDtypeStruct((N, D), jnp.float32),
    )(outp, denp, ssel, bias128, fc_W, fc_b2, g2, b2, gfc_W, gfc_b2)


def kernel(x, edge_index, W, att_src, att_dst, conv_bias, fc_W, fc_b,
           ln_gamma, ln_beta, gfc_W, gfc_b):
    f32 = jnp.float32
    w_cat = jnp.transpose(W, (1, 0, 2)).reshape(D, D)
    a_src_m = (jnp.eye(H, dtype=f32)[:, None, :]
               * att_src[:, :, None]).reshape(D, H)
    a_dst_m = (jnp.eye(H, dtype=f32)[:, None, :]
               * att_dst[:, :, None]).reshape(D, H)
    ssel = jnp.repeat(jnp.eye(H, dtype=f32), HD, axis=1)
    bias128 = conv_bias.reshape(1, D)

    loops = jnp.arange(N, dtype=jnp.int32)
    pad_i = jnp.arange(PAD, dtype=jnp.int32)
    src_all = jnp.concatenate(
        [edge_index[0], loops, N + (pad_i % NP)]).reshape(NW, NCH, B)
    dst_all = jnp.concatenate(
        [edge_index[1], loops, pad_i % N]).reshape(NW, NCH, B)

    hx_x, asx, adx, mg2 = _prep(x, w_cat, a_src_m, a_dst_m)
    outp, denp = _edges(hx_x, asx, adx, mg2, src_all, dst_all)
    return _tail(outp, denp, ssel, bias128, fc_W,
                 fc_b.reshape(1, D), ln_gamma.reshape(1, D),
                 ln_beta.reshape(1, D), gfc_W, gfc_b.reshape(1, D))


# speedup vs baseline: 301.2625x; 1.2307x over previous
import functools

import jax
import jax.numpy as jnp
from jax import lax
from jax.experimental import pallas as pl
from jax.experimental.pallas import tpu as pltpu
from jax.experimental.pallas import tpu_sc as plsc

N = 10000
E = 320000
D = 128
H = 8
HD = 16

NP = 8
NX = N + NP
A = E + N
NW = 32
B = 96
NCH = 108
C = NCH * B
A_PAD = NW * C
PAD = A_PAD - A
NA = 10240
TPT = NA // 16
ZR = 128


def _prep_body(x_ref, wc_ref, as_ref, ad_ref, hx_ref, asx_ref, adx_ref, mg_ref):
    hx = jnp.dot(x_ref[:], wc_ref[:], preferred_element_type=jnp.float32)
    a_s = jnp.dot(hx, as_ref[:], preferred_element_type=jnp.float32)
    a_d = jnp.dot(hx, ad_ref[:], preferred_element_type=jnp.float32)
    hx_ref[:] = jnp.concatenate([hx, jnp.zeros((NP, D), jnp.float32)], axis=0)
    neg = jnp.full((NP, 2 * H), -1e30, jnp.float32)
    z = jnp.zeros((N, H), jnp.float32)
    asx_ref[:] = jnp.concatenate(
        [jnp.concatenate([a_s, z], axis=1), neg], axis=0)
    adx_ref[:] = jnp.concatenate(
        [jnp.concatenate([a_d, z], axis=1), jnp.zeros((NP, 2 * H))], axis=0)
    m = jnp.max(a_s, axis=0) + jnp.max(a_d, axis=0)
    mg = jnp.where(m >= 0, m, 0.2 * m)
    mg_ref[:] = jnp.concatenate([mg, mg]).reshape(1, 2 * H)


def _prep(x, w_cat, a_src_m, a_dst_m):
    return pl.pallas_call(
        _prep_body,
        out_shape=[
            jax.ShapeDtypeStruct((NX, D), jnp.float32),
            jax.ShapeDtypeStruct((NX, 2 * H), jnp.float32),
            jax.ShapeDtypeStruct((NX, 2 * H), jnp.float32),
            jax.ShapeDtypeStruct((1, 2 * H), jnp.float32),
        ],
    )(x, w_cat, a_src_m, a_dst_m)


_MESH = plsc.VectorSubcoreMesh(core_axis_name="c", subcore_axis_name="s")


@functools.partial(
    pl.kernel,
    out_type=[
        jax.ShapeDtypeStruct((2, NA, D), jnp.float32),
        jax.ShapeDtypeStruct((2, NA, 2 * H), jnp.float32),
    ],
    mesh=_MESH,
    compiler_params=pltpu.CompilerParams(use_tc_tiling_on_sc=False),
    scratch_types=[
        [pltpu.VMEM((B,), jnp.int32) for _ in range(3)],
        [pltpu.VMEM((B,), jnp.int32) for _ in range(3)],
        [pltpu.VMEM((B, 2 * H), jnp.float32) for _ in range(2)],
        [pltpu.VMEM((B, 2 * H), jnp.float32) for _ in range(2)],
        [pltpu.VMEM((B, 2 * H), jnp.float32) for _ in range(2)],
        [pltpu.VMEM((B, D), jnp.float32) for _ in range(2)],
        pltpu.VMEM((16,), jnp.float32),
        [pltpu.SemaphoreType.DMA for _ in range(3)],
        [pltpu.SemaphoreType.DMA for _ in range(2)],
        [pltpu.SemaphoreType.DMA for _ in range(2)],
        [pltpu.SemaphoreType.DMA for _ in range(2)],
        pltpu.VMEM_SHARED((NA, D), jnp.float32),
        pltpu.VMEM_SHARED((NA, 2 * H), jnp.float32),
    ],
)
def _edges(hx_hbm, asx_hbm, adx_hbm, mg_hbm, src_hbm, dst_hbm,
           outp_hbm, denp_hbm,
           isr, idr, asb, adb, wb, hxb, mgv, si, sa, sd, sh,
           out_sh, den_sh):
    cid = lax.axis_index("c")
    sid = lax.axis_index("s")
    wid = sid * 2 + cid

    pltpu.sync_copy(mg_hbm.at[0], mgv)

    z16 = jnp.zeros((16,), jnp.float32)

    def zero_row(i, carry):
        for j in range(D // 16):
            hxb[0][i, pl.ds(16 * j, 16)] = z16
        wb[0][i, :] = z16
        return carry

    lax.fori_loop(0, B, zero_row, 0)
    for k in range(TPT // B):
        base = sid * TPT + k * B
        pltpu.sync_copy(hxb[0], out_sh.at[pl.ds(base, B)])
        pltpu.sync_copy(wb[0], den_sh.at[pl.ds(base, B)])
    plsc.subcore_barrier()

    mg = mgv[:]

    def issue_idx(j, p):
        pltpu.async_copy(src_hbm.at[wid, j], isr[p], si[p])
        pltpu.async_copy(dst_hbm.at[wid, j], idr[p], si[p])

    def wait_idx(j, p):
        pltpu.make_async_copy(src_hbm.at[wid, j], isr[p], si[p]).wait()
        pltpu.make_async_copy(dst_hbm.at[wid, j], idr[p], si[p]).wait()

    def issue_data(p3, p2):
        pltpu.async_copy(asx_hbm.at[isr[p3]], asb[p2], sa[p2])
        pltpu.async_copy(adx_hbm.at[idr[p3]], adb[p2], sd[p2])
        pltpu.async_copy(hx_hbm.at[isr[p3]], hxb[p2], sh[p2])

    def wait_data(p3, p2):
        pltpu.make_async_copy(asx_hbm.at[isr[p3]], asb[p2], sa[p2]).wait()
        pltpu.make_async_copy(adx_hbm.at[idr[p3]], adb[p2], sd[p2]).wait()
        pltpu.make_async_copy(hx_hbm.at[isr[p3]], hxb[p2], sh[p2]).wait()

    issue_idx(0, 0)
    wait_idx(0, 0)
    issue_data(0, 0)
    issue_idx(1, 1)

    def group(g, carry):
        for u in range(6):
            jj = g * 6 + u
            p2, p3 = u % 2, u % 3
            q2, q3 = (u + 1) % 2, (u + 1) % 3
            r3 = (u + 2) % 3

            @pl.when(jj + 1 < NCH)
            def _():
                wait_idx(jj + 1, q3)
                issue_data(q3, q2)

            @pl.when(jj + 2 < NCH)
            def _():
                issue_idx(jj + 2, r3)

            wait_data(p3, p2)

            @plsc.parallel_loop(0, B, unroll=4)
            def _compute(i):
                e0 = asb[p2][i, :] + adb[p2][i, :]
                wv = jnp.exp(jnp.where(e0 >= 0, e0, 0.2 * e0) - mg)
                wb[p2][i, :] = wv
                for h in range(H):
                    s = wv[h]
                    hxb[p2][i, pl.ds(16 * h, 16)] = (
                        hxb[p2][i, pl.ds(16 * h, 16)] * s)

            pltpu.sync_copy(wb[p2], den_sh.at[idr[p3]], add=True)
            pltpu.sync_copy(hxb[p2], out_sh.at[idr[p3]], add=True)
        return carry

    lax.fori_loop(0, NCH // 6, group, 0)
    plsc.subcore_barrier()

    base = sid * TPT
    pltpu.sync_copy(out_sh.at[pl.ds(base, TPT)],
                    outp_hbm.at[cid, pl.ds(base, TPT)])
    pltpu.sync_copy(den_sh.at[pl.ds(base, TPT)],
                    denp_hbm.at[cid, pl.ds(base, TPT)])


def _tail_body(outp_ref, denp_ref, ssel_ref, bias_ref, fcw_ref, fcb_ref,
               g_ref, b_ref, gfcw_ref, gfcb_ref, o_ref):
    num = outp_ref[0, 0:N, :] + outp_ref[1, 0:N, :]
    den8 = denp_ref[0, 0:N, 0:H] + denp_ref[1, 0:N, 0:H]
    d128 = jnp.dot(1.0 / (den8 + 1e-16), ssel_ref[:],
                   preferred_element_type=jnp.float32)
    xl = num * d128 + bias_ref[:]
    t = lax.dot_general(xl, fcw_ref[:], (((1,), (1,)), ((), ())),
                        preferred_element_type=jnp.float32) + fcb_ref[:]
    sa = jnp.where(t >= 0, t, 0.01 * t)
    sa = jnp.exp(sa - jnp.max(sa, axis=-1, keepdims=True))
    sa = sa / jnp.sum(sa, axis=-1, keepdims=True)
    xl = xl * sa
    xl = jnp.where(xl >= 0, xl, 0.2 * xl)
    xl = lax.dot_general(xl, fcw_ref[:], (((1,), (1,)), ((), ())),
                         preferred_element_type=jnp.float32) + fcb_ref[:]
    mu = jnp.mean(xl, axis=-1, keepdims=True)
    var = jnp.mean((xl - mu) ** 2, axis=-1, keepdims=True)
    xl = (xl - mu) / jnp.sqrt(var + 1e-5) * g_ref[:] + b_ref[:]
    nrm = jnp.sqrt(jnp.sum(xl * xl, axis=1, keepdims=True))
    xl = xl / jnp.maximum(nrm, 1e-12)
    xg = jnp.mean(xl, axis=0, keepdims=True)
    ga = lax.dot_general(xg, gfcw_ref[:], (((1,), (1,)), ((), ())),
                         preferred_element_type=jnp.float32) + gfcb_ref[:]
    ga = jnp.maximum(ga, 0.0)
    ga = jnp.exp(ga - jnp.max(ga, axis=-1, keepdims=True))
    ga = ga / jnp.sum(ga, axis=-1, keepdims=True)
    o_ref[:] = xl * ga


def _tail(outp, denp, ssel, bias128, fc_W, fc_b2, g2, b2, gfc_W, gfc_b2):
    return pl.pallas_call(
        _tail_body,
        out_shape=jax.ShapeDtypeStruct((N, D), jnp.float32),
    )(outp, denp, ssel, bias128, fc_W, fc_b2, g2, b2, gfc_W, gfc_b2)


def kernel(x, edge_index, W, att_src, att_dst, conv_bias, fc_W, fc_b,
           ln_gamma, ln_beta, gfc_W, gfc_b):
    f32 = jnp.float32
    w_cat = jnp.transpose(W, (1, 0, 2)).reshape(D, D)
    a_src_m = (jnp.eye(H, dtype=f32)[:, None, :]
               * att_src[:, :, None]).reshape(D, H)
    a_dst_m = (jnp.eye(H, dtype=f32)[:, None, :]
               * att_dst[:, :, None]).reshape(D, H)
    ssel = jnp.repeat(jnp.eye(H, dtype=f32), HD, axis=1)
    bias128 = conv_bias.reshape(1, D)

    loops = jnp.arange(N, dtype=jnp.int32)
    pad_i = jnp.arange(PAD, dtype=jnp.int32)
    src_all = jnp.concatenate(
        [edge_index[0], loops, N + (pad_i % NP)]).reshape(NW, NCH, B)
    dst_all = jnp.concatenate(
        [edge_index[1], loops, pad_i % N]).reshape(NW, NCH, B)

    hx_x, asx, adx, mg2 = _prep(x, w_cat, a_src_m, a_dst_m)
    outp, denp = _edges(hx_x, asx, adx, mg2, src_all, dst_all)
    return _tail(outp, denp, ssel, bias128, fc_W,
                 fc_b.reshape(1, D), ln_gamma.reshape(1, D),
                 ln_beta.reshape(1, D), gfc_W, gfc_b.reshape(1, D))
